# baseline (speedup 1.0000x reference)
import functools

import jax
import jax.numpy as jnp
from jax import lax
from jax.experimental import pallas as pl
from jax.experimental.pallas import tpu as pltpu
from jax.experimental.pallas import tpu_sc as plsc

F32 = jnp.float32
I32 = jnp.int32

N_NODES = 10000
N_GRAPHS = 500
NPG = 20
K = 8
EMB = 128
VOCAB = 150

NC, NS, LANES = 2, 16, 16
NW = NC * NS

N_PAD = 10240
E_PAD = 327680
N3_PAD = 512
E3_PAD = 8192
TAB_PAD = 160
SEL_PAD = 4096


@functools.lru_cache(maxsize=None)
def _mk_gather(n_out, d):
    per_tile = n_out // NW
    ch = 64
    mesh = plsc.VectorSubcoreMesh(core_axis_name="c", subcore_axis_name="s")

    @functools.partial(
        pl.kernel,
        out_type=jax.ShapeDtypeStruct((n_out, d), F32),
        mesh=mesh,
        compiler_params=pltpu.CompilerParams(needs_layout_passes=False),
        scratch_types=[
            pltpu.VMEM((ch,), I32),
            pltpu.VMEM((ch, d), F32),
            pltpu.SemaphoreType.DMA,
        ],
    )
    def gk(table_h, idx_h, out_h, idx_v, rows_v, sem):
        wid = lax.axis_index("c") * NS + lax.axis_index("s")
        for j in range(per_tile // ch):
            off = wid * per_tile + j * ch
            pltpu.sync_copy(idx_h.at[pl.ds(off, ch)], idx_v)
            pltpu.async_copy(table_h.at[idx_v], rows_v, sem).wait()
            pltpu.sync_copy(rows_v, out_h.at[pl.ds(off, ch)])

    return gk


@functools.lru_cache(maxsize=None)
def _mk_edge_agg(n_pad, e_pad, d):
    per_tile_e = e_pad // NW
    ch = 64 if per_tile_e % (64 * 8) == 0 else 32
    n_chunks = per_tile_e // ch
    assert n_chunks % 8 == 0
    sbc = 8
    n_sb = n_chunks // sbc
    rows_per_tile = n_pad // NS
    cs = min(ch, rows_per_tile)
    mesh = plsc.VectorSubcoreMesh(core_axis_name="c", subcore_axis_name="s")

    @functools.partial(
        pl.kernel,
        out_type=[
            jax.ShapeDtypeStruct((NC, n_pad, d), F32),
            jax.ShapeDtypeStruct((NW, n_pad), F32),
        ],
        mesh=mesh,
        compiler_params=pltpu.CompilerParams(needs_layout_passes=False),
        scratch_types=[
            pltpu.VMEM((n_pad,), I32),
            pltpu.VMEM((sbc, ch), I32),
            pltpu.VMEM((sbc, ch), I32),
            pltpu.VMEM((ch,), F32),
            pltpu.VMEM((2, ch, d), F32),
            pltpu.VMEM((n_pad,), F32),
            pltpu.VMEM_SHARED((n_pad, d), F32),
            pltpu.SemaphoreType.DMA,
            pltpu.SemaphoreType.DMA,
            pltpu.SemaphoreType.DMA,
            pltpu.SemaphoreType.DMA,
        ],
    )
    def ek(src_h, dst_h, eler_h, feat_h, num_h, den_h,
           eler_v, src_v, dst_v, w_v, rows_v, den_v, num_sh,
           sem0, sem1, ssem0, ssem1):
        cid = lax.axis_index("c")
        sid = lax.axis_index("s")
        wid = cid * NS + sid
        sems = (sem0, sem1)
        ssems = (ssem0, ssem1)

        def zrow(r, carry):
            for c in range(d // LANES):
                rows_v[0, r, pl.ds(c * LANES, LANES)] = jnp.zeros((LANES,), F32)
            return carry

        lax.fori_loop(0, ch, zrow, 0)

        def zden(i, carry):
            den_v[pl.ds(i * LANES, LANES)] = jnp.zeros((LANES,), F32)
            return carry

        lax.fori_loop(0, n_pad // LANES, zden, 0)

        row0 = sid * rows_per_tile
        for j in range(rows_per_tile // cs):
            pltpu.sync_copy(rows_v.at[0, pl.ds(0, cs)],
                            num_sh.at[pl.ds(row0 + j * cs, cs)])
        plsc.subcore_barrier()

        pltpu.sync_copy(eler_h, eler_v)

        base_row = wid * n_chunks

        def issue_gather(k):
            pltpu.async_copy(feat_h.at[src_v.at[k]], rows_v.at[k % 2],
                             sems[k % 2])

        def wait_gather(k):
            pltpu.make_async_copy(feat_h.at[src_v.at[k]], rows_v.at[k % 2],
                                  sems[k % 2]).wait()

        def issue_scatter(k):
            pltpu.async_copy(rows_v.at[k % 2], num_sh.at[dst_v.at[k]],
                             ssems[k % 2], add=True)

        def wait_scatter(k):
            pltpu.make_async_copy(rows_v.at[k % 2], num_sh.at[dst_v.at[k]],
                                  ssems[k % 2]).wait()

        def sb_body(sb, carry):
            @pl.when(sb > 0)
            def _():
                wait_scatter(sbc - 1)

            r0 = base_row + sb * sbc
            pltpu.sync_copy(src_h.at[pl.ds(r0, sbc)], src_v)
            pltpu.sync_copy(dst_h.at[pl.ds(r0, sbc)], dst_v)
            issue_gather(0)
            for k in range(sbc):
                if k + 1 < sbc:
                    if k >= 1:
                        wait_scatter(k - 1)
                    issue_gather(k + 1)
                for i in range(ch // LANES):
                    s = src_v[k, pl.ds(i * LANES, LANES)]
                    t = dst_v[k, pl.ds(i * LANES, LANES)]
                    elp = plsc.load_gather(eler_v, [s])
                    erp = plsc.load_gather(eler_v, [t])
                    elf = plsc.bitcast(elp & jnp.int32(-65536), F32)
                    erf = plsc.bitcast(erp << 16, F32)
                    ev = elf + erf
                    ev = jnp.where(ev > 0, ev, 0.2 * ev)
                    w = jnp.exp(ev)
                    w_v[pl.ds(i * LANES, LANES)] = w
                    plsc.addupdate_scatter(den_v, [t], w)
                wait_gather(k)

                @plsc.parallel_loop(0, ch, unroll=4)
                def scale_row(r):
                    wr = plsc.load_gather(w_v, [jnp.zeros((LANES,), I32) + r])
                    for c in range(d // LANES):
                        rows_v[k % 2, r, pl.ds(c * LANES, LANES)] = (
                            rows_v[k % 2, r, pl.ds(c * LANES, LANES)] * wr)

                issue_scatter(k)
            wait_scatter(sbc - 2)
            return carry

        lax.fori_loop(0, n_sb, sb_body, 0)
        wait_scatter(sbc - 1)

        pltpu.sync_copy(den_v, den_h.at[wid])
        plsc.subcore_barrier()
        for j in range(rows_per_tile // cs):
            pltpu.sync_copy(num_sh.at[pl.ds(row0 + j * cs, cs)],
                            num_h.at[cid, pl.ds(row0 + j * cs, cs)])

    return ek


@functools.lru_cache(maxsize=None)
def _mk_vocab_agg(n_pad, e_pad, v):
    per_tile_e = e_pad // NW
    ch = 128
    n_chunks = per_tile_e // ch
    assert n_chunks % 8 == 0
    sbc = 8
    n_sb = n_chunks // sbc
    qn = n_pad * v
    per_tile_q = qn // NS
    zb = 6400
    assert per_tile_q % zb == 0
    mesh = plsc.VectorSubcoreMesh(core_axis_name="c", subcore_axis_name="s")

    @functools.partial(
        pl.kernel,
        out_type=[jax.ShapeDtypeStruct((qn,), F32),
                  jax.ShapeDtypeStruct((qn,), F32)],
        mesh=mesh,
        compiler_params=pltpu.CompilerParams(needs_layout_passes=False),
        scratch_types=[
            pltpu.VMEM((n_pad,), I32),
            pltpu.VMEM((160,), F32),
            pltpu.VMEM((160,), F32),
            pltpu.VMEM((sbc, ch), I32),
            pltpu.VMEM((sbc, ch), I32),
            pltpu.VMEM((2, ch), F32),
            pltpu.VMEM((2, ch), I32),
            pltpu.VMEM((zb,), F32),
            pltpu.VMEM_SHARED((qn,), F32),
            pltpu.SemaphoreType.DMA,
            pltpu.SemaphoreType.DMA,
        ],
    )
    def qk(src_h, dst_h, h_h, elt_h, ert_h, q0_h, q1_h,
           h_v, elt_v, ert_v, src_v, dst_v, w_v, fl_v, z_v, q_sh,
           ssem0, ssem1):
        cid = lax.axis_index("c")
        sid = lax.axis_index("s")
        wid = cid * NS + sid
        ssems = (ssem0, ssem1)

        def zz(i, carry):
            z_v[pl.ds(i * LANES, LANES)] = jnp.zeros((LANES,), F32)
            return carry

        lax.fori_loop(0, zb // LANES, zz, 0)
        q0 = sid * per_tile_q
        for j in range(per_tile_q // zb):
            pltpu.sync_copy(z_v, q_sh.at[pl.ds(q0 + j * zb, zb)])
        plsc.subcore_barrier()
        pltpu.sync_copy(h_h, h_v)
        pltpu.sync_copy(elt_h, elt_v)
        pltpu.sync_copy(ert_h, ert_v)
        base_row = wid * n_chunks

        def wait_scatter(b):
            pltpu.make_async_copy(w_v.at[b], q_sh.at[fl_v.at[b]],
                                  ssems[b]).wait()

        def sb_body(sb, carry):
            @pl.when(sb > 0)
            def _():
                wait_scatter(0)
                wait_scatter(1)

            r0 = base_row + sb * sbc
            pltpu.sync_copy(src_h.at[pl.ds(r0, sbc)], src_v)
            pltpu.sync_copy(dst_h.at[pl.ds(r0, sbc)], dst_v)
            for k in range(sbc):
                b = k % 2
                if k >= 2:
                    wait_scatter(b)
                for i in range(ch // LANES):
                    s = src_v[k, pl.ds(i * LANES, LANES)]
                    t = dst_v[k, pl.ds(i * LANES, LANES)]
                    hs = plsc.load_gather(h_v, [s])
                    ht = plsc.load_gather(h_v, [t])
                    el = plsc.load_gather(elt_v, [hs])
                    er = plsc.load_gather(ert_v, [ht])
                    ev = el + er
                    ev = jnp.where(ev > 0, ev, 0.2 * ev)
                    w_v[b, pl.ds(i * LANES, LANES)] = jnp.exp(ev)
                    fl_v[b, pl.ds(i * LANES, LANES)] = t * v + hs
                pltpu.async_copy(w_v.at[b], q_sh.at[fl_v.at[b]], ssems[b],
                                 add=True)
            return carry

        lax.fori_loop(0, n_sb, sb_body, 0)
        wait_scatter(0)
        wait_scatter(1)
        plsc.subcore_barrier()

        @pl.when(cid == 0)
        def _():
            for j in range(per_tile_q // zb):
                pltpu.sync_copy(q_sh.at[pl.ds(q0 + j * zb, zb)],
                                q0_h.at[pl.ds(q0 + j * zb, zb)])

        @pl.when(cid == 1)
        def _():
            for j in range(per_tile_q // zb):
                pltpu.sync_copy(q_sh.at[pl.ds(q0 + j * zb, zb)],
                                q1_h.at[pl.ds(q0 + j * zb, zb)])

    return qk


def _tab_kernel(emb_ref, w1_ref, al_ref, ar_ref, o_ref, eler_ref):
    t1 = jax.nn.relu(emb_ref[...]) @ w1_ref[...]
    o_ref[...] = t1
    elt = jnp.sum(t1 * al_ref[...], axis=1)
    ert = jnp.sum(t1 * ar_ref[...], axis=1)
    eler_ref[...] = jnp.stack([elt, ert], axis=0)


def _pack_eler(el, er):
    eli = jax.lax.bitcast_convert_type(el, I32)
    eri = jax.lax.bitcast_convert_type(er, I32)
    return (eli & jnp.int32(-65536)) | ((eri >> 16) & jnp.int32(0xFFFF))


def _midq_kernel(q0_ref, q1_ref, t1_ref, b_ref, w_ref, al_ref, ar_ref,
                 feat_ref, eler_ref):
    q = q0_ref[...] + q1_ref[...]
    den = jnp.sum(q, axis=1)
    den = jnp.where(den == 0, 1.0, den)
    num = q @ t1_ref[...]
    out = jax.nn.relu(num / den[:, None] + b_ref[...])
    f = out @ w_ref[...]
    feat_ref[...] = f
    el = jnp.sum(f * al_ref[...], axis=1)
    er = jnp.sum(f * ar_ref[...], axis=1)
    eler_ref[...] = _pack_eler(el, er)[None, :]


def _bitonic_lanes(x):
    n = x.shape[1]
    liota = lax.broadcasted_iota(I32, x.shape, 1)
    k = 2
    while k <= n:
        j = k // 2
        while j >= 1:
            plo = pltpu.roll(x, n - j, axis=1)
            phi = pltpu.roll(x, j, axis=1)
            low = (liota & j) == 0
            partner = jnp.where(low, plo, phi)
            asc = (liota & k) == 0
            take_min = low == asc
            x = jnp.where(take_min, jnp.minimum(x, partner),
                          jnp.maximum(x, partner))
            j //= 2
        k *= 2
    return x


def _mid2_kernel(num_ref, den_ref, b_ref, wf_ref, bf_ref, xf_ref, nmax_ref):
    num = num_ref[0] + num_ref[1]
    den = jnp.sum(den_ref[...], axis=0)
    den = jnp.where(den == 0, 1.0, den)
    out = jax.nn.relu(num / den[:, None] + b_ref[...])
    xf = jax.nn.relu(out @ wf_ref[...] + bf_ref[...])
    xf_ref[...] = xf
    nmax_ref[...] = jnp.max(xf, axis=1)[None, :]


def _sort_kernel(x_ref, o_ref):
    o_ref[...] = _bitonic_lanes(x_ref[...])


def _topk_kernel(nm_ref, o_ref):
    cur = nm_ref[...]
    liota = lax.broadcasted_iota(I32, cur.shape, 1)
    cols = []
    for _ in range(K):
        m = jnp.max(cur, axis=1, keepdims=True)
        idx = jnp.min(jnp.where(cur == m, liota, 10 ** 9), axis=1,
                      keepdims=True)
        cols.append(idx)
        cur = jnp.where(liota == idx, -1e30, cur)
    idxs = jnp.concatenate(cols, axis=1)
    o_ref[...] = idxs + lax.broadcasted_iota(I32, idxs.shape, 0) * NPG


def _mm3_kernel(sel_ref, w3_ref, al_ref, ar_ref, feat_ref, eler_ref):
    f = sel_ref[...] @ w3_ref[...]
    feat_ref[...] = f
    el = jnp.sum(f * al_ref[...], axis=1)
    er = jnp.sum(f * ar_ref[...], axis=1)
    eler_ref[...] = _pack_eler(el, er)[None, :]


def _head_kernel(num_ref, den_ref, b_ref, wl_ref, bl_ref, wl1_ref, bl1_ref,
                 wc_ref, bc_ref, o_ref):
    num = num_ref[0] + num_ref[1]
    den = jnp.sum(den_ref[...], axis=0)
    den = jnp.where(den == 0, 1.0, den)
    x = jax.nn.relu(num / den[:, None] + b_ref[...])
    x = jax.nn.relu(x @ wl_ref[...] + bl_ref[...])
    x = jax.nn.relu(x @ wl1_ref[...] + bl1_ref[...])
    o_ref[...] = x @ wc_ref[...] + bc_ref[...]


def kernel(h, g_edge_index, fg_edge_index, emb, W1, al1, ar1, b1, W2, al2,
           ar2, b2, Wf, bf, W3, al3, ar3, b3, Wl, bl, Wl1, bl1, Wc, bc):
    dummy = N_PAD - 1
    h_pad = jnp.concatenate([h, jnp.zeros((N_PAD - N_NODES,), I32)])
    src = jnp.concatenate(
        [g_edge_index[0], jnp.full((E_PAD - g_edge_index.shape[1],), dummy, I32)])
    dst = jnp.concatenate(
        [g_edge_index[1], jnp.full((E_PAD - g_edge_index.shape[1],), dummy, I32)])
    dummy3 = N3_PAD - 1
    src3 = jnp.concatenate(
        [fg_edge_index[0], jnp.full((E3_PAD - fg_edge_index.shape[1],), dummy3, I32)])
    dst3 = jnp.concatenate(
        [fg_edge_index[1], jnp.full((E3_PAD - fg_edge_index.shape[1],), dummy3, I32)])

    t1, eltert = pl.pallas_call(
        _tab_kernel,
        out_shape=[jax.ShapeDtypeStruct((VOCAB, EMB), F32),
                   jax.ShapeDtypeStruct((2, VOCAB), F32)],
    )(emb, W1, al1[None, :], ar1[None, :])
    eltert_p = jnp.pad(eltert, ((0, 0), (0, 160 - VOCAB)))

    q0, q1 = _mk_vocab_agg(N_PAD, E_PAD, VOCAB)(
        src.reshape(-1, 128), dst.reshape(-1, 128), h_pad,
        eltert_p[0], eltert_p[1])
    q0 = q0.reshape(N_PAD, VOCAB)
    q1 = q1.reshape(N_PAD, VOCAB)

    blk = 1280
    grid8 = (N_PAD // blk,)
    feat2, eler2 = pl.pallas_call(
        _midq_kernel, grid=grid8,
        in_specs=[
            pl.BlockSpec((blk, VOCAB), lambda i: (i, 0)),
            pl.BlockSpec((blk, VOCAB), lambda i: (i, 0)),
            pl.BlockSpec((VOCAB, EMB), lambda i: (0, 0)),
            pl.BlockSpec((1, EMB), lambda i: (0, 0)),
            pl.BlockSpec((EMB, EMB), lambda i: (0, 0)),
            pl.BlockSpec((1, EMB), lambda i: (0, 0)),
            pl.BlockSpec((1, EMB), lambda i: (0, 0)),
        ],
        out_specs=[
            pl.BlockSpec((blk, EMB), lambda i: (i, 0)),
            pl.BlockSpec((1, blk), lambda i: (0, i)),
        ],
        out_shape=[jax.ShapeDtypeStruct((N_PAD, EMB), F32),
                   jax.ShapeDtypeStruct((1, N_PAD), I32)],
    )(q0, q1, t1, b1[None, :], W2, al2[None, :], ar2[None, :])

    num2, den2 = _mk_edge_agg(N_PAD, E_PAD, EMB)(
        src.reshape(-1, 64), dst.reshape(-1, 64), eler2.reshape(N_PAD), feat2)

    xf, nmax = pl.pallas_call(
        _mid2_kernel, grid=grid8,
        in_specs=[
            pl.BlockSpec((NC, blk, EMB), lambda i: (0, i, 0)),
            pl.BlockSpec((NW, blk), lambda i: (0, i)),
            pl.BlockSpec((1, EMB), lambda i: (0, 0)),
            pl.BlockSpec((EMB, EMB), lambda i: (0, 0)),
            pl.BlockSpec((1, EMB), lambda i: (0, 0)),
        ],
        out_specs=[
            pl.BlockSpec((blk, EMB), lambda i: (i, 0)),
            pl.BlockSpec((1, blk), lambda i: (0, i)),
        ],
        out_shape=[jax.ShapeDtypeStruct((N_PAD, EMB), F32),
                   jax.ShapeDtypeStruct((1, N_PAD), F32)],
    )(num2, den2, b2[None, :], Wf, bf[None, :])

    nm = nmax[0, :N_NODES].reshape(N_GRAPHS, NPG)
    ids = pl.pallas_call(
        _topk_kernel,
        out_shape=jax.ShapeDtypeStruct((N_GRAPHS, K), I32),
    )(nm)
    ids_flat = jnp.concatenate(
        [ids.reshape(-1), jnp.zeros((SEL_PAD - N_GRAPHS * K,), I32)])

    sel = _mk_gather(SEL_PAD, EMB)(xf, ids_flat)
    sblk = 1024
    sel = pl.pallas_call(
        _sort_kernel, grid=(SEL_PAD // sblk,),
        in_specs=[pl.BlockSpec((sblk, EMB), lambda i: (i, 0))],
        out_specs=pl.BlockSpec((sblk, EMB), lambda i: (i, 0)),
        out_shape=jax.ShapeDtypeStruct((SEL_PAD, EMB), F32),
    )(sel)
    sel1024 = sel[:N_GRAPHS * K].reshape(N_GRAPHS, K * EMB)
    sel1024 = jnp.concatenate(
        [sel1024, jnp.zeros((N3_PAD - N_GRAPHS, K * EMB), F32)])

    feat3, eler3 = pl.pallas_call(
        _mm3_kernel,
        out_shape=[jax.ShapeDtypeStruct((N3_PAD, EMB), F32),
                   jax.ShapeDtypeStruct((1, N3_PAD), I32)],
    )(sel1024, W3, al3[None, :], ar3[None, :])

    num3, den3 = _mk_edge_agg(N3_PAD, E3_PAD, EMB)(
        src3.reshape(-1, 32), dst3.reshape(-1, 32), eler3.reshape(N3_PAD),
        feat3)

    out = pl.pallas_call(
        _head_kernel,
        out_shape=jax.ShapeDtypeStruct((N3_PAD, 2), F32),
    )(num3, den3, b3[None, :], Wl, bl[None, :], Wl1, bl1[None, :], Wc,
      bc[None, :])
    return out[:N_GRAPHS].reshape(-1, 2)


# speedup vs baseline: 1.1182x; 1.1182x over previous
import functools

import jax
import jax.numpy as jnp
from jax import lax
from jax.experimental import pallas as pl
from jax.experimental.pallas import tpu as pltpu
from jax.experimental.pallas import tpu_sc as plsc

F32 = jnp.float32
I32 = jnp.int32

N_NODES = 10000
N_GRAPHS = 500
NPG = 20
K = 8
EMB = 128
VOCAB = 150

NC, NS, LANES = 2, 16, 16
NW = NC * NS

N_PAD = 10240
E_PAD = 327680
N3_PAD = 512
E3_PAD = 8192
SEL_PAD = 4096


@functools.lru_cache(maxsize=None)
def _mk_gather(n_out, d):
    per_tile = n_out // NW
    ch = 64
    mesh = plsc.VectorSubcoreMesh(core_axis_name="c", subcore_axis_name="s")

    @functools.partial(
        pl.kernel,
        out_type=jax.ShapeDtypeStruct((n_out, d), F32),
        mesh=mesh,
        compiler_params=pltpu.CompilerParams(needs_layout_passes=False),
        scratch_types=[
            pltpu.VMEM((ch,), I32),
            pltpu.VMEM((ch, d), F32),
            pltpu.SemaphoreType.DMA,
        ],
    )
    def gk(table_h, idx_h, out_h, idx_v, rows_v, sem):
        wid = lax.axis_index("c") * NS + lax.axis_index("s")
        for j in range(per_tile // ch):
            off = wid * per_tile + j * ch
            pltpu.sync_copy(idx_h.at[pl.ds(off, ch)], idx_v)
            pltpu.async_copy(table_h.at[idx_v], rows_v, sem).wait()
            pltpu.sync_copy(rows_v, out_h.at[pl.ds(off, ch)])

    return gk


@functools.lru_cache(maxsize=None)
def _mk_edge_agg(n_pad, e_pad, d):
    per_tile_e = e_pad // NW
    ch = 64
    n_chunks = per_tile_e // ch
    assert n_chunks % 2 == 0
    rows_per_tile = n_pad // NS
    cs = min(ch, rows_per_tile)
    mesh = plsc.VectorSubcoreMesh(core_axis_name="c", subcore_axis_name="s")

    @functools.partial(
        pl.kernel,
        out_type=[
            jax.ShapeDtypeStruct((NC, n_pad, d), F32),
            jax.ShapeDtypeStruct((NW, n_pad), F32),
        ],
        mesh=mesh,
        compiler_params=pltpu.CompilerParams(needs_layout_passes=False),
        scratch_types=[
            pltpu.VMEM((n_pad,), F32),
            pltpu.VMEM((n_pad,), F32),
            pltpu.VMEM((2, ch), I32),
            pltpu.VMEM((2, ch), I32),
            pltpu.VMEM((ch,), F32),
            pltpu.VMEM((2, ch, d), F32),
            pltpu.VMEM((n_pad,), F32),
            pltpu.VMEM_SHARED((n_pad, d), F32),
            pltpu.SemaphoreType.DMA,
            pltpu.SemaphoreType.DMA,
        ],
    )
    def ek(src_h, dst_h, el_h, er_h, feat_h, num_h, den_h,
           el_v, er_v, src_v, dst_v, w_v, rows_v, den_v, num_sh, sem0, sem1):
        cid = lax.axis_index("c")
        sid = lax.axis_index("s")
        wid = cid * NS + sid
        sems = (sem0, sem1)

        def zrow(r, carry):
            for c in range(d // LANES):
                rows_v[0, r, pl.ds(c * LANES, LANES)] = jnp.zeros((LANES,), F32)
            return carry

        lax.fori_loop(0, ch, zrow, 0)

        def zden(i, carry):
            den_v[pl.ds(i * LANES, LANES)] = jnp.zeros((LANES,), F32)
            return carry

        lax.fori_loop(0, n_pad // LANES, zden, 0)

        row0 = sid * rows_per_tile
        for j in range(rows_per_tile // cs):
            pltpu.sync_copy(rows_v.at[0, pl.ds(0, cs)],
                            num_sh.at[pl.ds(row0 + j * cs, cs)])
        plsc.subcore_barrier()

        pltpu.sync_copy(el_h, el_v)
        pltpu.sync_copy(er_h, er_v)

        base_e = wid * per_tile_e

        def fetch(b, g):
            off = base_e + g * ch
            pltpu.sync_copy(src_h.at[pl.ds(off, ch)], src_v.at[b])
            pltpu.sync_copy(dst_h.at[pl.ds(off, ch)], dst_v.at[b])
            pltpu.async_copy(feat_h.at[src_v.at[b]], rows_v.at[b], sems[b])

        def process(b):
            for i in range(ch // LANES):
                s = src_v[b, pl.ds(i * LANES, LANES)]
                t = dst_v[b, pl.ds(i * LANES, LANES)]
                ev = plsc.load_gather(el_v, [s]) + plsc.load_gather(er_v, [t])
                ev = jnp.where(ev > 0, ev, 0.2 * ev)
                w = jnp.exp(ev)
                w_v[pl.ds(i * LANES, LANES)] = w
                plsc.addupdate_scatter(den_v, [t], w)
            pltpu.make_async_copy(feat_h.at[src_v.at[b]], rows_v.at[b],
                                  sems[b]).wait()

            def scale_row(r, carry2):
                wr = plsc.load_gather(w_v, [jnp.zeros((LANES,), I32) + r])
                for c in range(d // LANES):
                    rows_v[b, r, pl.ds(c * LANES, LANES)] = (
                        rows_v[b, r, pl.ds(c * LANES, LANES)] * wr)
                return carry2

            lax.fori_loop(0, ch, scale_row, 0)
            pltpu.sync_copy(rows_v.at[b], num_sh.at[dst_v.at[b]], add=True)

        fetch(0, 0)

        def pair_body(p, carry):
            fetch(1, 2 * p + 1)
            process(0)

            @pl.when(p + 1 < n_chunks // 2)
            def _():
                fetch(0, 2 * p + 2)

            process(1)
            return carry

        lax.fori_loop(0, n_chunks // 2, pair_body, 0)

        pltpu.sync_copy(den_v, den_h.at[wid])
        plsc.subcore_barrier()
        for j in range(rows_per_tile // cs):
            pltpu.sync_copy(num_sh.at[pl.ds(row0 + j * cs, cs)],
                            num_h.at[cid, pl.ds(row0 + j * cs, cs)])

    return ek


@functools.lru_cache(maxsize=None)
def _mk_vocab_agg(n_pad, e_pad, v):
    per_tile_e = e_pad // NW
    ch = 128
    n_chunks = per_tile_e // ch
    qn = n_pad * v
    per_tile_q = qn // NS
    zb = 6400
    assert per_tile_q % zb == 0
    mesh = plsc.VectorSubcoreMesh(core_axis_name="c", subcore_axis_name="s")

    @functools.partial(
        pl.kernel,
        out_type=[jax.ShapeDtypeStruct((qn,), F32),
                  jax.ShapeDtypeStruct((qn,), F32)],
        mesh=mesh,
        compiler_params=pltpu.CompilerParams(needs_layout_passes=False),
        scratch_types=[
            pltpu.VMEM((n_pad,), I32),
            pltpu.VMEM((160,), F32),
            pltpu.VMEM((160,), F32),
            pltpu.VMEM((ch,), I32),
            pltpu.VMEM((ch,), I32),
            pltpu.VMEM((ch,), F32),
            pltpu.VMEM((ch,), I32),
            pltpu.VMEM((zb,), F32),
            pltpu.VMEM_SHARED((qn,), F32),
        ],
    )
    def qk(src_h, dst_h, h_h, elt_h, ert_h, q0_h, q1_h,
           h_v, elt_v, ert_v, src_v, dst_v, w_v, fl_v, z_v, q_sh):
        cid = lax.axis_index("c")
        sid = lax.axis_index("s")
        wid = cid * NS + sid

        def zz(i, carry):
            z_v[pl.ds(i * LANES, LANES)] = jnp.zeros((LANES,), F32)
            return carry

        lax.fori_loop(0, zb // LANES, zz, 0)
        q0 = sid * per_tile_q
        for j in range(per_tile_q // zb):
            pltpu.sync_copy(z_v, q_sh.at[pl.ds(q0 + j * zb, zb)])
        plsc.subcore_barrier()
        pltpu.sync_copy(h_h, h_v)
        pltpu.sync_copy(elt_h, elt_v)
        pltpu.sync_copy(ert_h, ert_v)
        base_e = wid * per_tile_e

        def chunk(g, carry):
            off = base_e + g * ch
            pltpu.sync_copy(src_h.at[pl.ds(off, ch)], src_v)
            pltpu.sync_copy(dst_h.at[pl.ds(off, ch)], dst_v)
            for i in range(ch // LANES):
                s = src_v[pl.ds(i * LANES, LANES)]
                t = dst_v[pl.ds(i * LANES, LANES)]
                hs = plsc.load_gather(h_v, [s])
                ht = plsc.load_gather(h_v, [t])
                el = plsc.load_gather(elt_v, [hs])
                er = plsc.load_gather(ert_v, [ht])
                ev = el + er
                ev = jnp.where(ev > 0, ev, 0.2 * ev)
                w_v[pl.ds(i * LANES, LANES)] = jnp.exp(ev)
                fl_v[pl.ds(i * LANES, LANES)] = t * v + hs
            pltpu.sync_copy(w_v, q_sh.at[fl_v], add=True)
            return carry

        lax.fori_loop(0, n_chunks, chunk, 0)
        plsc.subcore_barrier()

        @pl.when(cid == 0)
        def _():
            for j in range(per_tile_q // zb):
                pltpu.sync_copy(q_sh.at[pl.ds(q0 + j * zb, zb)],
                                q0_h.at[pl.ds(q0 + j * zb, zb)])

        @pl.when(cid == 1)
        def _():
            for j in range(per_tile_q // zb):
                pltpu.sync_copy(q_sh.at[pl.ds(q0 + j * zb, zb)],
                                q1_h.at[pl.ds(q0 + j * zb, zb)])

    return qk


def _tab_kernel(emb_ref, w1_ref, al_ref, ar_ref, o_ref, eler_ref):
    t1 = jax.nn.relu(emb_ref[...]) @ w1_ref[...]
    o_ref[...] = t1
    elt = jnp.sum(t1 * al_ref[...], axis=1)
    ert = jnp.sum(t1 * ar_ref[...], axis=1)
    eler_ref[...] = jnp.stack([elt, ert], axis=0)


def _midq_kernel(q0_ref, q1_ref, t1_ref, b_ref, w_ref, al_ref, ar_ref,
                 feat_ref, eler_ref):
    q = q0_ref[...] + q1_ref[...]
    den = jnp.sum(q, axis=1)
    den = jnp.where(den == 0, 1.0, den)
    num = q @ t1_ref[...]
    out = jax.nn.relu(num / den[:, None] + b_ref[...])
    f = out @ w_ref[...]
    feat_ref[...] = f
    el = jnp.sum(f * al_ref[...], axis=1)
    er = jnp.sum(f * ar_ref[...], axis=1)
    eler_ref[...] = jnp.stack([el, er], axis=0)


def _bitonic_lanes(x):
    n = x.shape[1]
    liota = lax.broadcasted_iota(I32, x.shape, 1)
    k = 2
    while k <= n:
        j = k // 2
        while j >= 1:
            plo = pltpu.roll(x, n - j, axis=1)
            phi = pltpu.roll(x, j, axis=1)
            low = (liota & j) == 0
            partner = jnp.where(low, plo, phi)
            asc = (liota & k) == 0
            take_min = low == asc
            x = jnp.where(take_min, jnp.minimum(x, partner),
                          jnp.maximum(x, partner))
            j //= 2
        k *= 2
    return x


def _mid2_kernel(num_ref, den_ref, b_ref, wf_ref, bf_ref, xf_ref, nmax_ref):
    num = num_ref[0] + num_ref[1]
    den = jnp.sum(den_ref[...], axis=0)
    den = jnp.where(den == 0, 1.0, den)
    out = jax.nn.relu(num / den[:, None] + b_ref[...])
    xf = jax.nn.relu(out @ wf_ref[...] + bf_ref[...])
    xf_ref[...] = xf
    nmax_ref[...] = jnp.max(xf, axis=1)[None, :]


def _sort_kernel(x_ref, o_ref):
    o_ref[...] = _bitonic_lanes(x_ref[...])


def _topk_kernel(nm_ref, o_ref):
    cur = nm_ref[...]
    liota = lax.broadcasted_iota(I32, cur.shape, 1)
    cols = []
    for _ in range(K):
        m = jnp.max(cur, axis=1, keepdims=True)
        idx = jnp.min(jnp.where(cur == m, liota, 10 ** 9), axis=1,
                      keepdims=True)
        cols.append(idx)
        cur = jnp.where(liota == idx, -1e30, cur)
    idxs = jnp.concatenate(cols, axis=1)
    o_ref[...] = idxs + lax.broadcasted_iota(I32, idxs.shape, 0) * NPG


def _mm3_kernel(sel_ref, w3_ref, al_ref, ar_ref, feat_ref, eler_ref):
    f = sel_ref[...] @ w3_ref[...]
    feat_ref[...] = f
    el = jnp.sum(f * al_ref[...], axis=1)
    er = jnp.sum(f * ar_ref[...], axis=1)
    eler_ref[...] = jnp.stack([el, er], axis=0)


def _head_kernel(num_ref, den_ref, b_ref, wl_ref, bl_ref, wl1_ref, bl1_ref,
                 wc_ref, bc_ref, o_ref):
    num = num_ref[0] + num_ref[1]
    den = jnp.sum(den_ref[...], axis=0)
    den = jnp.where(den == 0, 1.0, den)
    x = jax.nn.relu(num / den[:, None] + b_ref[...])
    x = jax.nn.relu(x @ wl_ref[...] + bl_ref[...])
    x = jax.nn.relu(x @ wl1_ref[...] + bl1_ref[...])
    o_ref[...] = x @ wc_ref[...] + bc_ref[...]


def kernel(h, g_edge_index, fg_edge_index, emb, W1, al1, ar1, b1, W2, al2,
           ar2, b2, Wf, bf, W3, al3, ar3, b3, Wl, bl, Wl1, bl1, Wc, bc):
    dummy = N_PAD - 1
    h_pad = jnp.concatenate([h, jnp.zeros((N_PAD - N_NODES,), I32)])
    src = jnp.concatenate(
        [g_edge_index[0], jnp.full((E_PAD - g_edge_index.shape[1],), dummy, I32)])
    dst = jnp.concatenate(
        [g_edge_index[1], jnp.full((E_PAD - g_edge_index.shape[1],), dummy, I32)])
    dummy3 = N3_PAD - 1
    src3 = jnp.concatenate(
        [fg_edge_index[0], jnp.full((E3_PAD - fg_edge_index.shape[1],), dummy3, I32)])
    dst3 = jnp.concatenate(
        [fg_edge_index[1], jnp.full((E3_PAD - fg_edge_index.shape[1],), dummy3, I32)])

    t1, eltert = pl.pallas_call(
        _tab_kernel,
        out_shape=[jax.ShapeDtypeStruct((VOCAB, EMB), F32),
                   jax.ShapeDtypeStruct((2, VOCAB), F32)],
    )(emb, W1, al1[None, :], ar1[None, :])
    eltert_p = jnp.pad(eltert, ((0, 0), (0, 160 - VOCAB)))

    q0, q1 = _mk_vocab_agg(N_PAD, E_PAD, VOCAB)(
        src, dst, h_pad, eltert_p[0], eltert_p[1])
    q0 = q0.reshape(N_PAD, VOCAB)
    q1 = q1.reshape(N_PAD, VOCAB)

    blk = 1280
    grid8 = (N_PAD // blk,)
    feat2, eler2 = pl.pallas_call(
        _midq_kernel, grid=grid8,
        in_specs=[
            pl.BlockSpec((blk, VOCAB), lambda i: (i, 0)),
            pl.BlockSpec((blk, VOCAB), lambda i: (i, 0)),
            pl.BlockSpec((VOCAB, EMB), lambda i: (0, 0)),
            pl.BlockSpec((1, EMB), lambda i: (0, 0)),
            pl.BlockSpec((EMB, EMB), lambda i: (0, 0)),
            pl.BlockSpec((1, EMB), lambda i: (0, 0)),
            pl.BlockSpec((1, EMB), lambda i: (0, 0)),
        ],
        out_specs=[
            pl.BlockSpec((blk, EMB), lambda i: (i, 0)),
            pl.BlockSpec((2, blk), lambda i: (0, i)),
        ],
        out_shape=[jax.ShapeDtypeStruct((N_PAD, EMB), F32),
                   jax.ShapeDtypeStruct((2, N_PAD), F32)],
    )(q0, q1, t1, b1[None, :], W2, al2[None, :], ar2[None, :])

    num2, den2 = _mk_edge_agg(N_PAD, E_PAD, EMB)(
        src, dst, eler2[0], eler2[1], feat2)

    xf, nmax = pl.pallas_call(
        _mid2_kernel, grid=grid8,
        in_specs=[
            pl.BlockSpec((NC, blk, EMB), lambda i: (0, i, 0)),
            pl.BlockSpec((NW, blk), lambda i: (0, i)),
            pl.BlockSpec((1, EMB), lambda i: (0, 0)),
            pl.BlockSpec((EMB, EMB), lambda i: (0, 0)),
            pl.BlockSpec((1, EMB), lambda i: (0, 0)),
        ],
        out_specs=[
            pl.BlockSpec((blk, EMB), lambda i: (i, 0)),
            pl.BlockSpec((1, blk), lambda i: (0, i)),
        ],
        out_shape=[jax.ShapeDtypeStruct((N_PAD, EMB), F32),
                   jax.ShapeDtypeStruct((1, N_PAD), F32)],
    )(num2, den2, b2[None, :], Wf, bf[None, :])

    nm = nmax[0, :N_NODES].reshape(N_GRAPHS, NPG)
    ids = pl.pallas_call(
        _topk_kernel,
        out_shape=jax.ShapeDtypeStruct((N_GRAPHS, K), I32),
    )(nm)
    ids_flat = jnp.concatenate(
        [ids.reshape(-1), jnp.zeros((SEL_PAD - N_GRAPHS * K,), I32)])

    sel = _mk_gather(SEL_PAD, EMB)(xf, ids_flat)
    sblk = 1024
    sel = pl.pallas_call(
        _sort_kernel, grid=(SEL_PAD // sblk,),
        in_specs=[pl.BlockSpec((sblk, EMB), lambda i: (i, 0))],
        out_specs=pl.BlockSpec((sblk, EMB), lambda i: (i, 0)),
        out_shape=jax.ShapeDtypeStruct((SEL_PAD, EMB), F32),
    )(sel)
    sel1024 = sel[:N_GRAPHS * K].reshape(N_GRAPHS, K * EMB)
    sel1024 = jnp.concatenate(
        [sel1024, jnp.zeros((N3_PAD - N_GRAPHS, K * EMB), F32)])

    feat3, eler3 = pl.pallas_call(
        _mm3_kernel,
        out_shape=[jax.ShapeDtypeStruct((N3_PAD, EMB), F32),
                   jax.ShapeDtypeStruct((2, N3_PAD), F32)],
    )(sel1024, W3, al3[None, :], ar3[None, :])

    num3, den3 = _mk_edge_agg(N3_PAD, E3_PAD, EMB)(
        src3, dst3, eler3[0], eler3[1], feat3)

    out = pl.pallas_call(
        _head_kernel,
        out_shape=jax.ShapeDtypeStruct((N3_PAD, 2), F32),
    )(num3, den3, b3[None, :], Wl, bl[None, :], Wl1, bl1[None, :], Wc,
      bc[None, :])
    return out[:N_GRAPHS].reshape(-1, 2)
